# Initial kernel scaffold; baseline (speedup 1.0000x reference)
import jax
import jax.numpy as jnp
from jax.experimental import pallas as pl


def kernel(tokens, tables):
    raise NotImplementedError("write your pallas kernel here")


# speedup vs baseline: 3.1328x; 3.1328x over previous
import functools

import jax
import jax.numpy as jnp
from jax import lax
from jax.experimental import pallas as pl
from jax.experimental.pallas import tpu as pltpu
from jax.experimental.pallas import tpu_sc as plsc

_CHUNK = 128
_LANES = 16


def _sc_lookup(tokens, table_flat, B, C, T, V, D):
    n_workers = 32
    halves = n_workers // B
    t_span = T // halves
    n_chunks = t_span // _CHUNK

    mesh = plsc.VectorSubcoreMesh(core_axis_name="c", subcore_axis_name="s")

    @functools.partial(
        pl.kernel,
        mesh=mesh,
        out_type=jax.ShapeDtypeStruct((B, T, C, D), jnp.float32),
        scratch_types=[
            pltpu.VMEM((_CHUNK,), jnp.int32),
            pltpu.VMEM((_CHUNK, D), jnp.float32),
            pltpu.SemaphoreType.DMA,
        ],
        compiler_params=pltpu.CompilerParams(use_tc_tiling_on_sc=False),
    )
    def body(tokens_hbm, table_hbm, out_hbm, idx_v, rows_v, sem):
        wid = lax.axis_index("s") * 2 + lax.axis_index("c")
        b = wid // halves
        t_base = (wid % halves) * t_span

        def step(it, carry):
            i = it // n_chunks
            t0 = t_base + (it % n_chunks) * _CHUNK
            pltpu.sync_copy(tokens_hbm.at[b, i, pl.ds(t0, _CHUNK)], idx_v)
            off = i * V
            for j in range(_CHUNK // _LANES):
                sl = pl.ds(j * _LANES, _LANES)
                idx_v[sl] = idx_v[sl] + off
            pltpu.async_copy(table_hbm.at[idx_v], rows_v, sem).wait()
            pltpu.sync_copy(rows_v, out_hbm.at[b, pl.ds(t0, _CHUNK), i])
            return carry

        lax.fori_loop(0, C * n_chunks, step, 0)

    return body(tokens, table_flat)


def kernel(tokens, tables):
    B, C, T = tokens.shape
    C2, V, D = tables.shape
    assert C == C2
    table_flat = tables.reshape(C * V, D)
    out = _sc_lookup(tokens.astype(jnp.int32), table_flat, B, C, T, V, D)
    return out.reshape(B, T, C * D)


# speedup vs baseline: 3.7765x; 1.2055x over previous
import functools

import jax
import jax.numpy as jnp
from jax import lax
from jax.experimental import pallas as pl
from jax.experimental.pallas import tpu as pltpu
from jax.experimental.pallas import tpu_sc as plsc

_CHUNK = 128
_LANES = 16


def _sc_lookup(tokens, table_flat, B, C, T, V, D):
    n_workers = 32
    halves = n_workers // B
    t_span = T // halves
    n_chunks = t_span // _CHUNK
    n_it = C * n_chunks

    mesh = plsc.VectorSubcoreMesh(core_axis_name="c", subcore_axis_name="s")

    @functools.partial(
        pl.kernel,
        mesh=mesh,
        out_type=jax.ShapeDtypeStruct((B, T, C, D), jnp.float32),
        scratch_types=[
            pltpu.VMEM((C, t_span), jnp.int32),
            pltpu.VMEM((2, _CHUNK, D), jnp.float32),
            pltpu.SemaphoreType.DMA,
            pltpu.SemaphoreType.DMA,
            pltpu.SemaphoreType.DMA,
            pltpu.SemaphoreType.DMA,
        ],
        compiler_params=pltpu.CompilerParams(use_tc_tiling_on_sc=False),
    )
    def body(tokens_hbm, table_hbm, out_hbm, idx_v, rows_v, g0, g1, w0, w1):
        wid = lax.axis_index("s") * 2 + lax.axis_index("c")
        b = wid // halves
        t_base = (wid % halves) * t_span
        g_sem = (g0, g1)
        w_sem = (w0, w1)

        pltpu.sync_copy(tokens_hbm.at[b, :, pl.ds(t_base, t_span)], idx_v)

        def add_off(i, carry):
            off = i * V
            for j in range(t_span // _LANES):
                sl = pl.ds(j * _LANES, _LANES)
                idx_v[i, sl] = idx_v[i, sl] + off
            return carry

        lax.fori_loop(1, C, add_off, 0)

        def idx_slice(it):
            return idx_v.at[it // n_chunks, pl.ds((it % n_chunks) * _CHUNK, _CHUNK)]

        def gather(it, p):
            return pltpu.make_async_copy(
                table_hbm.at[idx_slice(it)], rows_v.at[p], g_sem[p]
            )

        def write(it, p):
            t0 = t_base + (it % n_chunks) * _CHUNK
            return pltpu.make_async_copy(
                rows_v.at[p], out_hbm.at[b, pl.ds(t0, _CHUNK), it // n_chunks], w_sem[p]
            )

        def stage(it, p, q):
            @pl.when(it >= 1)
            def _():
                write(it - 1, q).wait()

            @pl.when(it + 1 < n_it)
            def _():
                gather(it + 1, q).start()

            gather(it, p).wait()
            write(it, p).start()

        gather(0, 0).start()

        def pair(it2, carry):
            stage(2 * it2, 0, 1)
            stage(2 * it2 + 1, 1, 0)
            return carry

        lax.fori_loop(0, n_it // 2, pair, 0)
        write(n_it - 1, 1).wait()

    return body(tokens, table_flat)


def kernel(tokens, tables):
    B, C, T = tokens.shape
    C2, V, D = tables.shape
    assert C == C2
    table_flat = tables.reshape(C * V, D)
    out = _sc_lookup(tokens.astype(jnp.int32), table_flat, B, C, T, V, D)
    return out.reshape(B, T, C * D)
